# baseline (speedup 1.0000x reference)
import functools

import jax
import jax.numpy as jnp
from jax import lax
from jax.experimental import pallas as pl
from jax.experimental.pallas import tpu as pltpu
from jax.experimental.pallas import tpu_sc as plsc

VOCAB = 1000
DIM = 64
NUM_CORES = 2
NUM_SUBCORES = 16
NW = NUM_CORES * NUM_SUBCORES
IDX_CAP = 128
NBUF = 2


def _make_sc_gather(B: int, F: int):
    rows_per_w = B // NW
    f_hi = F - IDX_CAP if F > IDX_CAP else 0

    mesh = plsc.VectorSubcoreMesh(core_axis_name="c", subcore_axis_name="s")

    @functools.partial(
        pl.kernel,
        mesh=mesh,
        compiler_params=pltpu.CompilerParams(use_tc_tiling_on_sc=False),
        out_type=jax.ShapeDtypeStruct((B, F, DIM), jnp.float32),
        scratch_types=[
            pltpu.VMEM((rows_per_w, F), jnp.int32),
            pltpu.VMEM((NBUF, F, DIM), jnp.float32),
            pltpu.VMEM_SHARED((VOCAB, DIM), jnp.float32),
            pltpu.SemaphoreType.DMA((NBUF,)),
            pltpu.SemaphoreType.DMA((NBUF,)),
        ],
    )
    def k(idx_hbm, table_hbm, out_hbm, idx_v, bufs, table_sh, gsems, wsems):
        cid = lax.axis_index("c")
        sid = lax.axis_index("s")
        wid = sid * NUM_CORES + cid
        base = wid * rows_per_w

        @pl.when(sid == 0)
        def _():
            pltpu.sync_copy(table_hbm, table_sh)

        pltpu.sync_copy(idx_hbm.at[pl.ds(base, rows_per_w)], idx_v)
        plsc.subcore_barrier()

        def gather(r, b):
            pltpu.async_copy(
                table_sh.at[idx_v.at[r, pl.ds(0, IDX_CAP)]],
                bufs.at[b, pl.ds(0, IDX_CAP)],
                gsems.at[b],
            )
            if f_hi:
                pltpu.async_copy(
                    table_sh.at[idx_v.at[r, pl.ds(IDX_CAP, f_hi)]],
                    bufs.at[b, pl.ds(IDX_CAP, f_hi)],
                    gsems.at[b],
                )

        def gather_wait(b):
            pltpu.make_async_copy(
                table_sh.at[idx_v.at[0, pl.ds(0, F)]], bufs.at[b], gsems.at[b]
            ).wait()

        def write(r, b):
            pltpu.async_copy(bufs.at[b], out_hbm.at[base + r], wsems.at[b])

        def write_wait(b):
            pltpu.make_async_copy(
                out_hbm.at[base], bufs.at[b], wsems.at[b]
            ).wait()

        for b in range(NBUF):
            gather(b, b)

        def body(i, carry):
            rg = i * NBUF
            for b in range(NBUF):
                gather_wait(b)
                write(rg + b, b)
            for b in range(NBUF):
                write_wait(b)
                gather(rg + NBUF + b, b)
            return carry

        lax.fori_loop(0, rows_per_w // NBUF - 1, body, 0)

        rg = rows_per_w - NBUF
        for b in range(NBUF):
            gather_wait(b)
            write(rg + b, b)
        for b in range(NBUF):
            write_wait(b)

    return k


def kernel(img_rep_tokens, table):
    b, f = img_rep_tokens.shape
    return _make_sc_gather(b, f)(img_rep_tokens, table)


# speedup vs baseline: 1.7647x; 1.7647x over previous
import functools

import jax
import jax.numpy as jnp
from jax import lax
from jax.experimental import pallas as pl
from jax.experimental.pallas import tpu as pltpu
from jax.experimental.pallas import tpu_sc as plsc

VOCAB = 1000
DIM = 64
LANE = 128
NUM_CORES = 2
NUM_SUBCORES = 16
NW = NUM_CORES * NUM_SUBCORES
CHUNK = 128
NBUF = 4


def _make_sc_gather(n_total: int):
    per_w = n_total // NW
    n_chunks = per_w // CHUNK
    n_groups = n_chunks // NBUF

    mesh = plsc.VectorSubcoreMesh(core_axis_name="c", subcore_axis_name="s")

    @functools.partial(
        pl.kernel,
        mesh=mesh,
        compiler_params=pltpu.CompilerParams(use_tc_tiling_on_sc=False),
        out_type=jax.ShapeDtypeStruct((n_total, LANE), jnp.float32),
        scratch_types=[
            pltpu.VMEM((n_chunks, CHUNK), jnp.int32),
            pltpu.VMEM((NBUF, CHUNK, LANE), jnp.float32),
            pltpu.VMEM_SHARED((VOCAB, LANE), jnp.float32),
            pltpu.SemaphoreType.DMA((NBUF,)),
            pltpu.SemaphoreType.DMA((NBUF,)),
        ],
    )
    def k(idx_hbm, table_hbm, out_hbm, idx_v, bufs, table_sh, gsems, wsems):
        cid = lax.axis_index("c")
        sid = lax.axis_index("s")
        wid = sid * NUM_CORES + cid
        base = wid * per_w

        @pl.when(sid == 0)
        def _():
            pltpu.sync_copy(table_hbm, table_sh)

        pltpu.sync_copy(idx_hbm.at[wid], idx_v)
        plsc.subcore_barrier()

        def gather(j, b):
            pltpu.async_copy(table_sh.at[idx_v.at[j]], bufs.at[b], gsems.at[b])

        def gather_wait(b):
            pltpu.make_async_copy(
                table_sh.at[idx_v.at[0]], bufs.at[b], gsems.at[b]
            ).wait()

        def write(j, b):
            pltpu.async_copy(
                bufs.at[b], out_hbm.at[pl.ds(base + j * CHUNK, CHUNK)], wsems.at[b]
            )

        def write_wait(b):
            pltpu.make_async_copy(
                out_hbm.at[pl.ds(base, CHUNK)], bufs.at[b], wsems.at[b]
            ).wait()

        for b in range(NBUF):
            gather(b, b)

        def body(i, carry):
            jg = i * NBUF
            for b in range(NBUF):
                gather_wait(b)
                write(jg + b, b)
            for b in range(NBUF):
                write_wait(b)
                gather(jg + NBUF + b, b)
            return carry

        lax.fori_loop(0, n_groups - 1, body, 0)

        jg = (n_groups - 1) * NBUF
        for b in range(NBUF):
            gather_wait(b)
            write(jg + b, b)
        for b in range(NBUF):
            write_wait(b)

    return k


def kernel(img_rep_tokens, table):
    b, f = img_rep_tokens.shape
    n = b * f
    idx3d = img_rep_tokens.reshape(NW, n // (NW * CHUNK), CHUNK)
    table_p = jnp.pad(table, ((0, 0), (0, LANE - DIM)))
    out = _make_sc_gather(n)(idx3d, table_p)
    return out[:, :DIM].reshape(b, f, DIM)


# speedup vs baseline: 2.2382x; 1.2683x over previous
import functools

import jax
import jax.numpy as jnp
from jax import lax
from jax.experimental import pallas as pl
from jax.experimental.pallas import tpu as pltpu
from jax.experimental.pallas import tpu_sc as plsc

VOCAB = 1000
DIM = 64
LANE = 128
NUM_CORES = 2
NUM_SUBCORES = 16
NW = NUM_CORES * NUM_SUBCORES
CHUNK = 128
NBUF = 4


def _make_sc_gather(n_total: int):
    per_w = n_total // NW
    n_chunks = per_w // CHUNK
    n_groups = n_chunks // NBUF

    mesh = plsc.VectorSubcoreMesh(core_axis_name="c", subcore_axis_name="s")

    @functools.partial(
        pl.kernel,
        mesh=mesh,
        compiler_params=pltpu.CompilerParams(use_tc_tiling_on_sc=False),
        out_type=jax.ShapeDtypeStruct((n_total, LANE), jnp.float32),
        scratch_types=[
            pltpu.VMEM((n_chunks, CHUNK), jnp.int32),
            pltpu.VMEM((NBUF, CHUNK, DIM), jnp.float32),
            pltpu.VMEM_SHARED((VOCAB, DIM), jnp.float32),
            pltpu.SemaphoreType.DMA((NBUF,)),
            pltpu.SemaphoreType.DMA((NBUF,)),
        ],
    )
    def k(idx_hbm, table_hbm, out_hbm, idx_v, bufs, table_sh, gsems, wsems):
        cid = lax.axis_index("c")
        sid = lax.axis_index("s")
        wid = sid * NUM_CORES + cid
        base = wid * per_w

        @pl.when(sid == 0)
        def _():
            pltpu.sync_copy(table_hbm, table_sh)

        pltpu.sync_copy(idx_hbm.at[wid], idx_v)
        plsc.subcore_barrier()

        def gather(j, b):
            pltpu.async_copy(table_sh.at[idx_v.at[j]], bufs.at[b], gsems.at[b])

        def gather_wait(b):
            pltpu.make_async_copy(
                table_sh.at[idx_v.at[0]], bufs.at[b], gsems.at[b]
            ).wait()

        def write(j, b):
            pltpu.async_copy(
                bufs.at[b],
                out_hbm.at[pl.ds(base + j * CHUNK, CHUNK), pl.ds(0, DIM)],
                wsems.at[b],
            )

        def write_wait(b):
            pltpu.make_async_copy(
                out_hbm.at[pl.ds(base, CHUNK), pl.ds(0, DIM)], bufs.at[b], wsems.at[b]
            ).wait()

        for b in range(NBUF):
            gather(b, b)

        def body(i, carry):
            jg = i * NBUF
            for b in range(NBUF):
                gather_wait(b)
                write(jg + b, b)
            for b in range(NBUF):
                write_wait(b)
                gather(jg + NBUF + b, b)
            return carry

        lax.fori_loop(0, n_groups - 1, body, 0)

        jg = (n_groups - 1) * NBUF
        for b in range(NBUF):
            gather_wait(b)
            write(jg + b, b)
        for b in range(NBUF):
            write_wait(b)

    return k


def kernel(img_rep_tokens, table):
    b, f = img_rep_tokens.shape
    n = b * f
    idx3d = img_rep_tokens.reshape(NW, n // (NW * CHUNK), CHUNK)
    out = _make_sc_gather(n)(idx3d, table)
    return out[:, :DIM].reshape(b, f, DIM)


# speedup vs baseline: 2.2483x; 1.0045x over previous
import functools

import jax
import jax.numpy as jnp
from jax import lax
from jax.experimental import pallas as pl
from jax.experimental.pallas import tpu as pltpu
from jax.experimental.pallas import tpu_sc as plsc

VOCAB = 1000
DIM = 64
LANE = 128
NUM_CORES = 2
NUM_SUBCORES = 16
NW = NUM_CORES * NUM_SUBCORES
CHUNK = 128
NBUF = 6


def _make_sc_gather(n_total: int):
    per_w = n_total // NW
    n_chunks = per_w // CHUNK
    n_groups = n_chunks // NBUF

    mesh = plsc.VectorSubcoreMesh(core_axis_name="c", subcore_axis_name="s")

    @functools.partial(
        pl.kernel,
        mesh=mesh,
        compiler_params=pltpu.CompilerParams(use_tc_tiling_on_sc=False),
        out_type=jax.ShapeDtypeStruct((n_total, LANE), jnp.float32),
        scratch_types=[
            pltpu.VMEM((n_chunks, CHUNK), jnp.int32),
            pltpu.VMEM((NBUF, CHUNK, DIM), jnp.float32),
            pltpu.VMEM_SHARED((VOCAB, DIM), jnp.float32),
            pltpu.SemaphoreType.DMA((NBUF,)),
            pltpu.SemaphoreType.DMA((NBUF,)),
        ],
    )
    def k(idx_hbm, table_hbm, out_hbm, idx_v, bufs, table_sh, gsems, wsems):
        cid = lax.axis_index("c")
        sid = lax.axis_index("s")
        wid = sid * NUM_CORES + cid
        base = wid * per_w

        @pl.when(sid == 0)
        def _():
            pltpu.sync_copy(table_hbm, table_sh)

        pltpu.sync_copy(idx_hbm.at[wid], idx_v)
        plsc.subcore_barrier()

        def gather(j, b):
            pltpu.async_copy(table_sh.at[idx_v.at[j]], bufs.at[b], gsems.at[b])

        def gather_wait(b):
            pltpu.make_async_copy(
                table_sh.at[idx_v.at[0]], bufs.at[b], gsems.at[b]
            ).wait()

        def write(j, b):
            pltpu.async_copy(
                bufs.at[b],
                out_hbm.at[pl.ds(base + j * CHUNK, CHUNK), pl.ds(0, DIM)],
                wsems.at[b],
            )

        def write_wait(b):
            pltpu.make_async_copy(
                out_hbm.at[pl.ds(base, CHUNK), pl.ds(0, DIM)], bufs.at[b], wsems.at[b]
            ).wait()

        for b in range(NBUF):
            gather(b, b)

        def body(i, carry):
            jg = i * NBUF
            for b in range(NBUF):
                gather_wait(b)
                write(jg + b, b)
            for b in range(NBUF):
                write_wait(b)
                gather(jg + NBUF + b, b)
            return carry

        lax.fori_loop(0, n_groups - 1, body, 0)

        jg = (n_groups - 1) * NBUF
        for b in range(NBUF):
            gather_wait(b)
            write(jg + b, b)
        for b in range(NBUF):
            write_wait(b)

    return k


def kernel(img_rep_tokens, table):
    b, f = img_rep_tokens.shape
    n = b * f
    idx3d = img_rep_tokens.reshape(NW, n // (NW * CHUNK), CHUNK)
    out = _make_sc_gather(n)(idx3d, table)
    return out[:, :DIM].reshape(b, f, DIM)
